# baseline (speedup 1.0000x reference)
import functools

import jax
import jax.numpy as jnp
from jax import lax
from jax.experimental import pallas as pl
from jax.experimental.pallas import tpu as pltpu
from jax.experimental.pallas import tpu_sc as plsc

DIM = 64
LANES = 16
NUM_CORES = 2
NUM_SUBCORES = 16
NUM_WORKERS = NUM_CORES * NUM_SUBCORES
CHUNK = 128


def _sc_partial_scores(user_idx, pos_idx, neg_idx, user_table, item_table,
                       batch):
    b_per_w = batch // NUM_WORKERS
    n_chunks = b_per_w // CHUNK

    mesh = plsc.VectorSubcoreMesh(core_axis_name="c", subcore_axis_name="s")

    @functools.partial(
        pl.kernel,
        mesh=mesh,
        out_type=jax.ShapeDtypeStruct((batch, LANES), jnp.float32),
        compiler_params=pltpu.CompilerParams(use_tc_tiling_on_sc=False),
        scratch_types=[
            pltpu.VMEM((n_chunks, CHUNK), jnp.int32),
            pltpu.VMEM((n_chunks, CHUNK), jnp.int32),
            pltpu.VMEM((n_chunks, CHUNK), jnp.int32),
            pltpu.VMEM((b_per_w, DIM), jnp.float32),
            pltpu.VMEM((b_per_w, DIM), jnp.float32),
            pltpu.VMEM((b_per_w, DIM), jnp.float32),
            pltpu.VMEM((b_per_w, LANES), jnp.float32),
            pltpu.SemaphoreType.DMA,
        ],
    )
    def sc_kernel(user_hbm, pos_hbm, neg_hbm, ut_hbm, it_hbm, out_hbm,
                  idx_u, idx_p, idx_n, u_v, p_v, n_v, o_v, sem):
        wid = lax.axis_index("s") * NUM_CORES + lax.axis_index("c")
        base = wid * b_per_w

        copies = []
        for j in range(n_chunks):
            off = base + j * CHUNK
            pltpu.sync_copy(user_hbm.at[wid, j], idx_u.at[j])
            pltpu.sync_copy(pos_hbm.at[wid, j], idx_p.at[j])
            pltpu.sync_copy(neg_hbm.at[wid, j], idx_n.at[j])
            del off
            dst = pl.ds(j * CHUNK, CHUNK)
            copies.append(pltpu.async_copy(ut_hbm.at[idx_u.at[j]],
                                           u_v.at[dst], sem))
            copies.append(pltpu.async_copy(it_hbm.at[idx_p.at[j]],
                                           p_v.at[dst], sem))
            copies.append(pltpu.async_copy(it_hbm.at[idx_n.at[j]],
                                           n_v.at[dst], sem))
        for c in copies:
            c.wait()

        def body(i, carry):
            acc = jnp.zeros((LANES,), jnp.float32)
            for c in range(DIM // LANES):
                sl = pl.ds(c * LANES, LANES)
                uu = u_v[i, sl]
                pp = p_v[i, sl]
                nn = n_v[i, sl]
                acc = acc + uu * (pp - nn)
            o_v[i, :] = acc
            return carry

        lax.fori_loop(0, b_per_w, body, 0)

        pltpu.sync_copy(o_v, out_hbm.at[pl.ds(base, b_per_w)])

    u3 = user_idx.reshape(NUM_WORKERS, n_chunks, CHUNK)
    p3 = pos_idx.reshape(NUM_WORKERS, n_chunks, CHUNK)
    n3 = neg_idx.reshape(NUM_WORKERS, n_chunks, CHUNK)
    return sc_kernel(u3, p3, n3, user_table, item_table)


def _tc_loss_body(w_ref, o_ref):
    w = w_ref[...]
    tmp = jnp.sum(w, axis=1)
    bpr = jnp.maximum(-tmp, 0.0) + jnp.log1p(jnp.exp(-jnp.abs(tmp)))
    o_ref[0, 0] = jnp.sum(bpr)


def kernel(user, pos, neg, user_table, item_table):
    batch = user.shape[0]
    partial = _sc_partial_scores(
        user.astype(jnp.int32), pos.astype(jnp.int32), neg.astype(jnp.int32),
        user_table, item_table, batch)
    loss = pl.pallas_call(
        _tc_loss_body,
        out_shape=jax.ShapeDtypeStruct((1, 1), jnp.float32),
        in_specs=[pl.BlockSpec(memory_space=pltpu.VMEM)],
        out_specs=pl.BlockSpec(memory_space=pltpu.SMEM),
    )(partial)
    return loss[0, 0]
